# baseline (speedup 1.0000x reference)
import jax
import jax.numpy as jnp
from jax import lax
from jax.experimental import pallas as pl
from jax.experimental.pallas import tpu as pltpu
from jax.experimental.pallas import tpu_sc as plsc

BATCH = 16384
EMBED = 32
NUM_CORES = 2
NUM_SUBCORES = 16
NUM_WORKERS = NUM_CORES * NUM_SUBCORES
B_PER_W = BATCH // NUM_WORKERS
CHUNK = 128
NUM_CHUNKS = B_PER_W // CHUNK


def _make_gather_kernel():
    mesh = plsc.VectorSubcoreMesh(core_axis_name="c", subcore_axis_name="s")
    row_t = jax.ShapeDtypeStruct((BATCH, EMBED), jnp.float32)

    def sc_gather(users, movies, user_table, movie_table):
        @pl.kernel(
            out_type=(row_t, row_t),
            mesh=mesh,
            compiler_params=pltpu.CompilerParams(use_tc_tiling_on_sc=False),
            scratch_types=[
                pltpu.VMEM((B_PER_W,), jnp.int32),
                pltpu.VMEM((B_PER_W,), jnp.int32),
                pltpu.VMEM((B_PER_W, EMBED), jnp.float32),
                pltpu.VMEM((B_PER_W, EMBED), jnp.float32),
                pltpu.SemaphoreType.DMA,
            ],
        )
        def k(u_idx_hbm, m_idx_hbm, ut_hbm, mt_hbm, uo_hbm, mo_hbm,
              ui_v, mi_v, ur_v, mr_v, sem):
            wid = lax.axis_index("s") * NUM_CORES + lax.axis_index("c")
            base = wid * B_PER_W
            pltpu.sync_copy(u_idx_hbm.at[pl.ds(base, B_PER_W)], ui_v)
            pltpu.sync_copy(m_idx_hbm.at[pl.ds(base, B_PER_W)], mi_v)
            copies = []
            for c in range(NUM_CHUNKS):
                sl = pl.ds(c * CHUNK, CHUNK)
                copies.append(pltpu.async_copy(
                    ut_hbm.at[ui_v.at[sl]], ur_v.at[sl], sem))
                copies.append(pltpu.async_copy(
                    mt_hbm.at[mi_v.at[sl]], mr_v.at[sl], sem))
            for cp in copies:
                cp.wait()
            pltpu.sync_copy(ur_v, uo_hbm.at[pl.ds(base, B_PER_W)])
            pltpu.sync_copy(mr_v, mo_hbm.at[pl.ds(base, B_PER_W)])

        return k(users, movies, user_table, movie_table)

    return sc_gather


_sc_gather = _make_gather_kernel()


def _dot_body(ug_ref, mg_ref, wu_ref, wm_ref, b_ref, o_ref):
    s = jnp.sum(ug_ref[...] * wu_ref[...], axis=1, keepdims=True)
    s += jnp.sum(mg_ref[...] * wm_ref[...], axis=1, keepdims=True)
    o_ref[...] = s + b_ref[0, 0]


def kernel(users, movies, user_table, movie_table, fc1_w, fc1_b):
    users = users.astype(jnp.int32)
    movies = movies.astype(jnp.int32)
    ug, mg = _sc_gather(users, movies, user_table, movie_table)
    wu = fc1_w[:, :EMBED]
    wm = fc1_w[:, EMBED:]
    b = fc1_b.reshape(1, 1)
    out = pl.pallas_call(
        _dot_body,
        out_shape=jax.ShapeDtypeStruct((BATCH, 1), jnp.float32),
    )(ug, mg, wu, wm, b)
    return out
